# baseline (speedup 1.0000x reference)
import jax
import jax.numpy as jnp
from jax.experimental import pallas as pl
from jax.experimental.pallas import tpu as pltpu

B = 16
H_NUM = 3
W_NUM = 9
GRID = 20
DIM = 1024
L = 64
T = H_NUM * W_NUM + 1


def _body(it_ref, local_ref, row_ref, col_ref, img_ref, typ_ref, out_ref):
    b = pl.program_id(0)
    it = it_ref[b]
    img = jnp.where(it == 0, img_ref[0, 0, :], img_ref[1, 0, :])
    bias = row_ref[0, 0, :] + col_ref[0, 0, :] + typ_ref[0, 0, :] + img
    out_ref[...] = local_ref[...] + bias[None, None, :]


def kernel(image_time, local_pos, row_embed, col_embed, image_embed, type_embed):
    it32 = image_time.astype(jnp.int32)
    row3 = row_embed.reshape(GRID, 1, DIM)
    col3 = col_embed.reshape(GRID, 1, DIM)
    img3 = image_embed.reshape(2, 1, DIM)
    typ3 = type_embed.reshape(2, 1, DIM)

    def row_idx(b, t):
        return (jnp.where(t == T - 1, GRID - 1, t // W_NUM), 0, 0)

    def col_idx(b, t):
        return (jnp.where(t == T - 1, GRID - 1, t % W_NUM), 0, 0)

    def typ_idx(b, t):
        return (jnp.where(t == T - 1, 1, 0), 0, 0)

    out = pl.pallas_call(
        _body,
        grid=(B, T),
        in_specs=[
            pl.BlockSpec(memory_space=pltpu.SMEM),
            pl.BlockSpec((1, L, DIM), lambda b, t: (0, 0, 0)),
            pl.BlockSpec((1, 1, DIM), row_idx),
            pl.BlockSpec((1, 1, DIM), col_idx),
            pl.BlockSpec((2, 1, DIM), lambda b, t: (0, 0, 0)),
            pl.BlockSpec((1, 1, DIM), typ_idx),
        ],
        out_specs=pl.BlockSpec((1, L, DIM), lambda b, t: (b, t, 0)),
        out_shape=jax.ShapeDtypeStruct((B, T * L, DIM), jnp.float32),
    )(it32, local_pos, row3, col3, img3, typ3)
    return out


# speedup vs baseline: 6.2389x; 6.2389x over previous
import jax
import jax.numpy as jnp
from jax.experimental import pallas as pl
from jax.experimental.pallas import tpu as pltpu

B = 16
H_NUM = 3
W_NUM = 9
GRID = 20
DIM = 1024
L = 64
T = H_NUM * W_NUM + 1


def _body(it_ref, local_ref, row_ref, col_ref, img_ref, typ_ref, out_ref):
    b = pl.program_id(0)
    it = it_ref[b]
    img = jnp.where(it == 0, img_ref[0, :], img_ref[1, :])
    tmp = local_ref[0] + img[None, :]
    for t in range(T):
        y = GRID - 1 if t == T - 1 else t // W_NUM
        x = GRID - 1 if t == T - 1 else t % W_NUM
        m = 1 if t == T - 1 else 0
        comb = row_ref[y, :] + col_ref[x, :] + typ_ref[m, :]
        out_ref[0, t * L:(t + 1) * L, :] = tmp + comb[None, :]


def kernel(image_time, local_pos, row_embed, col_embed, image_embed, type_embed):
    it32 = image_time.astype(jnp.int32)
    out = pl.pallas_call(
        _body,
        grid=(B,),
        in_specs=[
            pl.BlockSpec(memory_space=pltpu.SMEM),
            pl.BlockSpec((1, L, DIM), lambda b: (0, 0, 0)),
            pl.BlockSpec((GRID, DIM), lambda b: (0, 0)),
            pl.BlockSpec((GRID, DIM), lambda b: (0, 0)),
            pl.BlockSpec((2, DIM), lambda b: (0, 0)),
            pl.BlockSpec((2, DIM), lambda b: (0, 0)),
        ],
        out_specs=pl.BlockSpec((1, T * L, DIM), lambda b: (b, 0, 0)),
        out_shape=jax.ShapeDtypeStruct((B, T * L, DIM), jnp.float32),
    )(it32, local_pos, row_embed, col_embed, image_embed, type_embed)
    return out
